# Initial kernel scaffold; baseline (speedup 1.0000x reference)
import jax
import jax.numpy as jnp
from jax.experimental import pallas as pl


def kernel(variables, factors, senders, receivers, W_msg, b_msg, W_comb, b_comb):
    raise NotImplementedError("write your pallas kernel here")


# speedup vs baseline: 7.0856x; 7.0856x over previous
import functools

import jax
import jax.numpy as jnp
from jax import lax
from jax.experimental import pallas as pl
from jax.experimental.pallas import tpu as pltpu
from jax.experimental.pallas import tpu_sc as plsc

NC = 2
NS = 16
NW = NC * NS
L = 16


def _pre_mm_kernel(v_ref, f_ref, w1_ref, w2_ref, b_ref, a_out, b_out):
    a_out[...] = (
        jnp.dot(v_ref[...], w1_ref[...], preferred_element_type=jnp.float32)
        + b_ref[...]
    )
    b_out[...] = jnp.dot(f_ref[...], w2_ref[...], preferred_element_type=jnp.float32)


def _comb_mm_kernel(v_ref, p0_ref, p1_ref, wc1_ref, wc2_ref, b_ref, o_ref):
    agg = p0_ref[...] + p1_ref[...]
    h = (
        jnp.dot(v_ref[...], wc1_ref[...], preferred_element_type=jnp.float32)
        + jnp.dot(agg, wc2_ref[...], preferred_element_type=jnp.float32)
        + b_ref[...]
    )
    o_ref[...] = v_ref[...] + jnp.maximum(h, 0.0)


def _make_sc_edge_kernel(n_pad, d, e, chunk, idx_block):
    per_w = e // NW
    n_chunks = per_w // chunk
    n_blocks = n_chunks // idx_block
    rows_per_tile = n_pad // NS

    mesh = plsc.VectorSubcoreMesh(core_axis_name="c", subcore_axis_name="s")

    @functools.partial(
        pl.kernel,
        out_type=jax.ShapeDtypeStruct((NC, n_pad, d), jnp.float32),
        mesh=mesh,
        scratch_types=[
            pltpu.VMEM((idx_block, chunk), jnp.int32),
            pltpu.VMEM((idx_block, chunk), jnp.int32),
            pltpu.VMEM((chunk, d), jnp.float32),
            pltpu.VMEM((chunk, d), jnp.float32),
            pltpu.VMEM_SHARED((n_pad, d), jnp.float32),
            pltpu.SemaphoreType.DMA,
            pltpu.SemaphoreType.DMA,
        ],
    )
    def sc_edge(a_hbm, b_hbm, snd_hbm, rcv_hbm, zero_hbm, out_hbm,
                snd_v, rcv_v, a_v, b_v, agg_sh, sem_a, sem_b):
        c = lax.axis_index("c")
        s = lax.axis_index("s")
        wid = c * NS + s
        own = pl.ds(s * rows_per_tile, rows_per_tile)

        pltpu.sync_copy(zero_hbm.at[own], agg_sh.at[own])
        plsc.subcore_barrier()

        def block_body(g, carry0):
            pltpu.sync_copy(snd_hbm.at[wid, g], snd_v)
            pltpu.sync_copy(rcv_hbm.at[wid, g], rcv_v)

            def chunk_body(k, carry):
                idx_a = snd_v.at[k]
                idx_b = rcv_v.at[k]
                cp_a = pltpu.async_copy(a_hbm.at[idx_a], a_v, sem_a)
                cp_b = pltpu.async_copy(b_hbm.at[idx_b], b_v, sem_b)
                cp_a.wait()
                cp_b.wait()

                def row_body(i, carry2):
                    for j in range(d // L):
                        sl = pl.ds(j * L, L)
                        a_v[i, sl] = jnp.maximum(a_v[i, sl] + b_v[i, sl], 0.0)
                    return carry2

                lax.fori_loop(0, chunk, row_body, 0, unroll=False)
                pltpu.sync_copy(a_v, agg_sh.at[idx_a], add=True)
                return carry

            lax.fori_loop(0, idx_block, chunk_body, 0, unroll=False)
            return carry0

        lax.fori_loop(0, n_blocks, block_body, 0, unroll=False)
        plsc.subcore_barrier()

        pltpu.sync_copy(agg_sh.at[own], out_hbm.at[c, own])

    return sc_edge


def kernel(variables, factors, senders, receivers, W_msg, b_msg, W_comb, b_comb):
    n_var, d = variables.shape
    e = senders.shape[0]

    w1 = W_msg[:d]
    w2 = W_msg[d:]
    wc1 = W_comb[:d]
    wc2 = W_comb[d:]
    bm = b_msg.reshape(1, d)
    bc = b_comb.reshape(1, d)

    blk = 2000
    grid = (n_var // blk,)
    row_spec = pl.BlockSpec((blk, d), lambda i: (i, 0))
    w_spec = pl.BlockSpec((d, d), lambda i: (0, 0))
    b_spec = pl.BlockSpec((1, d), lambda i: (0, 0))

    a_mat, b_mat = pl.pallas_call(
        _pre_mm_kernel,
        grid=grid,
        in_specs=[row_spec, row_spec, w_spec, w_spec, b_spec],
        out_specs=[row_spec, row_spec],
        out_shape=[
            jax.ShapeDtypeStruct((n_var, d), jnp.float32),
            jax.ShapeDtypeStruct((factors.shape[0], d), jnp.float32),
        ],
    )(variables, factors, w1, w2, bm)

    chunk = 80
    idx_block = 25
    per_w = e // NW
    n_chunks = per_w // chunk
    n_blocks = n_chunks // idx_block
    snd = senders.reshape(NW, n_blocks, idx_block, chunk)
    rcv = receivers.reshape(NW, n_blocks, idx_block, chunk)
    n_pad = -(-n_var // (NS * 8)) * (NS * 8)
    zeros = jnp.zeros((n_pad, d), jnp.float32)

    partial = _make_sc_edge_kernel(n_pad, d, e, chunk, idx_block)(
        a_mat, b_mat, snd, rcv, zeros)
    partial = partial[:, :n_var, :]

    out = pl.pallas_call(
        _comb_mm_kernel,
        grid=grid,
        in_specs=[row_spec, row_spec, row_spec, w_spec, w_spec, b_spec],
        out_specs=row_spec,
        out_shape=jax.ShapeDtypeStruct((n_var, d), jnp.float32),
    )(variables, partial[0], partial[1], wc1, wc2, bc)

    return out
